# baseline (speedup 1.0000x reference)
import functools
import math

import jax
import jax.numpy as jnp
from jax import lax
from jax.experimental import pallas as pl
from jax.experimental.pallas import tpu as pltpu
from jax.experimental.pallas import tpu_sc as plsc

D_MODEL = 64
SCALE = math.sqrt(D_MODEL)
NC, NS, L = 2, 16, 16
NW = NC * NS
B = 4096 * 200
ROWS_PER_W = B // NW
K = 8
CH = K * 128
NCHUNK = ROWS_PER_W // CH
IDX_ROWS = B // 128

_mesh = plsc.VectorSubcoreMesh(
    core_axis_name="c", subcore_axis_name="s", num_cores=NC, num_subcores=NS
)


@functools.partial(
    pl.kernel,
    mesh=_mesh,
    out_type=jax.ShapeDtypeStruct((B, D_MODEL), jnp.float32),
    scratch_types=[
        pltpu.VMEM((K, 128), jnp.int32),
        pltpu.VMEM((CH, D_MODEL), jnp.float32),
        pltpu.SemaphoreType.DMA,
    ],
    compiler_params=pltpu.CompilerParams(use_tc_tiling_on_sc=False),
)
def _emb_kernel(idx_hbm, lut_hbm, out_hbm, idx_v, rows_v, gsem):
    wid = lax.axis_index("s") * NC + lax.axis_index("c")
    idx_row0 = wid * (ROWS_PER_W // 128)
    out_row0 = wid * ROWS_PER_W

    def chunk_body(c, _):
        pltpu.sync_copy(idx_hbm.at[pl.ds(idx_row0 + c * K, K)], idx_v)
        descs = []
        for j in range(K):
            descs.append(
                pltpu.async_copy(
                    lut_hbm.at[idx_v.at[j]],
                    rows_v.at[pl.ds(j * 128, 128)],
                    gsem,
                )
            )
        for d in descs:
            d.wait()

        def scale_row(r, _):
            for q in range(D_MODEL // L):
                sl = pl.ds(q * L, L)
                rows_v[r, sl] = rows_v[r, sl] * SCALE
            return _

        lax.fori_loop(0, CH, scale_row, 0, unroll=4)

        pltpu.sync_copy(rows_v, out_hbm.at[pl.ds(out_row0 + c * CH, CH)])
        return _

    lax.fori_loop(0, NCHUNK, chunk_body, 0)


def kernel(x, lut):
    idx = x.astype(jnp.int32).reshape(IDX_ROWS, 128)
    out = _emb_kernel(idx, lut)
    return out.reshape(x.shape[0], x.shape[1], D_MODEL)


# speedup vs baseline: 1.4121x; 1.4121x over previous
import functools
import math

import jax
import jax.numpy as jnp
from jax import lax
from jax.experimental import pallas as pl
from jax.experimental.pallas import tpu as pltpu
from jax.experimental.pallas import tpu_sc as plsc

D_MODEL = 64
VOCAB = 1000000
T_DIM = 200
B_DIM = 4096
SCALE = math.sqrt(D_MODEL)
NC, NS, L = 2, 16, 16
D_PER_CORE = D_MODEL // NC
B_PER_SUB = B_DIM // NS
TG = 8
NG = T_DIM // TG

_mesh = plsc.VectorSubcoreMesh(
    core_axis_name="c", subcore_axis_name="s", num_cores=NC, num_subcores=NS
)


@functools.partial(
    pl.kernel,
    mesh=_mesh,
    out_type=jax.ShapeDtypeStruct((T_DIM, D_MODEL, B_DIM), jnp.float32),
    scratch_types=[
        pltpu.VMEM((2, T_DIM, 128), jnp.int32),
        pltpu.VMEM((TG, B_PER_SUB), jnp.float32),
        pltpu.VMEM_SHARED((VOCAB,), jnp.float32),
        pltpu.SemaphoreType.DMA,
    ],
)
def _emb_kernel(xt_hbm, lut_hbm, out_hbm, idx_res, vals, row_sh, gsem):
    c = lax.axis_index("c")
    s = lax.axis_index("s")
    b0 = s * B_PER_SUB

    for h in range(2):
        pltpu.sync_copy(xt_hbm.at[:, pl.ds(b0 + h * 128, 128)], idx_res.at[h])

    def d_body(d, _):
        d_global = c * D_PER_CORE + d
        plsc.subcore_barrier()

        @pl.when(s == 0)
        def _stage_row():
            pltpu.sync_copy(lut_hbm.at[d_global], row_sh)

        plsc.subcore_barrier()

        def g_body(g, _):
            t0 = g * TG
            descs = []
            for tt in range(TG):
                for h in range(2):
                    descs.append(
                        pltpu.async_copy(
                            row_sh.at[idx_res.at[h, t0 + tt]],
                            vals.at[tt, pl.ds(h * 128, 128)],
                            gsem,
                        )
                    )
            for dd in descs:
                dd.wait()

            for tt in range(TG):
                for q in range(B_PER_SUB // L):
                    sl = pl.ds(q * L, L)
                    vals[tt, sl] = vals[tt, sl] * SCALE

            pltpu.sync_copy(
                vals, out_hbm.at[pl.ds(t0, TG), d_global, pl.ds(b0, B_PER_SUB)]
            )
            return _

        lax.fori_loop(0, NG, g_body, 0)
        return _

    lax.fori_loop(0, D_PER_CORE, d_body, 0)


def kernel(x, lut):
    xt = x.astype(jnp.int32).T
    lut_t = lut.T
    out_t = _emb_kernel(xt, lut_t)
    return out_t.transpose(2, 0, 1)


# speedup vs baseline: 1.7935x; 1.2701x over previous
import functools
import math

import jax
import jax.numpy as jnp
from jax import lax
from jax.experimental import pallas as pl
from jax.experimental.pallas import tpu as pltpu
from jax.experimental.pallas import tpu_sc as plsc

D_MODEL = 64
VOCAB = 1000000
T_DIM = 200
B_DIM = 4096
SCALE = math.sqrt(D_MODEL)
NC, NS, L = 2, 16, 16
D_PER_CORE = D_MODEL // NC
B_PER_SUB = B_DIM // NS
TG = 8
NG = T_DIM // TG

_mesh = plsc.VectorSubcoreMesh(
    core_axis_name="c", subcore_axis_name="s", num_cores=NC, num_subcores=NS
)


@functools.partial(
    pl.kernel,
    mesh=_mesh,
    out_type=jax.ShapeDtypeStruct((T_DIM, D_MODEL, B_DIM), jnp.float32),
    scratch_types=[
        pltpu.VMEM((2, T_DIM, 128), jnp.int32),
        pltpu.VMEM((2, TG, B_PER_SUB), jnp.float32),
        pltpu.VMEM_SHARED((VOCAB,), jnp.float32),
        pltpu.SemaphoreType.DMA,
        pltpu.SemaphoreType.DMA,
        pltpu.SemaphoreType.DMA,
        pltpu.SemaphoreType.DMA,
    ],
)
def _emb_kernel(
    xt_hbm, lut_hbm, out_hbm, idx_res, vals, row_sh, gsem0, gsem1, ssem0, ssem1
):
    c = lax.axis_index("c")
    s = lax.axis_index("s")
    b0 = s * B_PER_SUB

    for h in range(2):
        pltpu.sync_copy(xt_hbm.at[:, pl.ds(b0 + h * 128, 128)], idx_res.at[h])

    def out_slice(g, d_global):
        return out_hbm.at[pl.ds(g * TG, TG), d_global, pl.ds(b0, B_PER_SUB)]

    def fire_gathers(g, buf, gsem):
        t0 = g * TG
        for tt in range(TG):
            for h in range(2):
                pltpu.async_copy(
                    row_sh.at[idx_res.at[h, t0 + tt]],
                    vals.at[buf, tt, pl.ds(h * 128, 128)],
                    gsem,
                )

    def drain(hbm_side, vmem_buf, sem):
        pltpu.make_async_copy(hbm_side, vmem_buf, sem).wait()

    def scale(buf):
        for tt in range(TG):
            for q in range(B_PER_SUB // L):
                sl = pl.ds(q * L, L)
                vals[buf, tt, sl] = vals[buf, tt, sl] * SCALE

    def d_body(d, _):
        d_global = c * D_PER_CORE + d
        plsc.subcore_barrier()

        @pl.when(s == 0)
        def _stage_row():
            pltpu.sync_copy(lut_hbm.at[d_global], row_sh)

        plsc.subcore_barrier()

        fire_gathers(0, 0, gsem0)

        def stage(g, buf, nbuf, gsem_b, gsem_n, ssem_b, ssem_n):
            @pl.when(g + 1 < NG)
            def _prefetch():
                @pl.when(g >= 1)
                def _wait_prev_store():
                    drain(out_slice(g - 1, d_global), vals.at[nbuf], ssem_n)

                fire_gathers(g + 1, nbuf, gsem_n)

            drain(out_slice(g, d_global), vals.at[buf], gsem_b)
            scale(buf)
            pltpu.async_copy(vals.at[buf], out_slice(g, d_global), ssem_b)

        def g_body(g, _):
            stage(2 * g, 0, 1, gsem0, gsem1, ssem0, ssem1)
            stage(2 * g + 1, 1, 0, gsem1, gsem0, ssem1, ssem0)
            return _

        lax.fori_loop(0, NG // 2, g_body, 0)
        if NG % 2:
            stage(NG - 1, 0, 1, gsem0, gsem1, ssem0, ssem1)
        drain(out_slice(NG - 2, d_global), vals.at[(NG - 2) % 2],
              ssem0 if (NG - 2) % 2 == 0 else ssem1)
        drain(out_slice(NG - 1, d_global), vals.at[(NG - 1) % 2],
              ssem0 if (NG - 1) % 2 == 0 else ssem1)
        return _

    lax.fori_loop(0, D_PER_CORE, d_body, 0)


def kernel(x, lut):
    xt = x.astype(jnp.int32).T
    lut_t = lut.T
    out_t = _emb_kernel(xt, lut_t)
    return out_t.transpose(2, 0, 1)
